# baseline (speedup 1.0000x reference)
import functools

import jax
import jax.numpy as jnp
from jax import lax
from jax.experimental import pallas as pl
from jax.experimental.pallas import tpu as pltpu
from jax.experimental.pallas import tpu_sc as plsc

K = 64
LANES = 16
NWORKERS = 32


def _lane_splat(vec, k):
    return lax.gather(
        vec, jnp.full((LANES, 1), k, jnp.int32),
        lax.GatherDimensionNumbers(offset_dims=(), collapsed_slice_dims=(0,),
                                   start_index_map=(0,)),
        (1,), mode=lax.GatherScatterMode.PROMISE_IN_BOUNDS)


def _keys_of(h):
    u = lax.bitcast_convert_type(h, jnp.uint32)
    return u ^ jnp.where(h < 0, jnp.uint32(0xFFFFFFFF), jnp.uint32(0x80000000))


def _enc_body(x_ref, ae_ref, be_ref, bd_ref, h_ref, tau_ref, gm_ref, *, nw):
    w = pl.program_id(0)
    xc = x_ref[...] - bd_ref[...]
    h = lax.dot_general(
        xc, ae_ref[...],
        dimension_numbers=(((1,), (1,)), ((), ())),
        preferred_element_type=jnp.float32,
    ) + be_ref[...]
    h_ref[...] = h

    @pl.when(w == 0)
    def _init():
        gm_ref[...] = h

    @pl.when(w > 0)
    def _acc():
        gm_ref[...] = jnp.maximum(gm_ref[...], h)

    @pl.when(w == nw - 1)
    def _threshold():
        gm = gm_ref[...]
        keys = _keys_of(gm)
        t = jnp.zeros((keys.shape[0], 1), jnp.uint32)
        for b in range(31, -1, -1):
            cand = t | jnp.uint32(1 << b)
            cnt = jnp.sum((keys >= cand).astype(jnp.int32), axis=1,
                          keepdims=True)
            t = jnp.where(cnt >= K, cand, t)
        sel = keys >= t
        tau_ref[...] = jnp.min(jnp.where(sel, gm, jnp.inf), axis=1,
                               keepdims=True)


def _encode(x, Ae, be, bd):
    b, dimin = x.shape
    width = Ae.shape[0]
    wb = 256
    nw = width // wb
    return pl.pallas_call(
        functools.partial(_enc_body, nw=nw),
        grid=(nw,),
        in_specs=[
            pl.BlockSpec((b, dimin), lambda w: (0, 0)),
            pl.BlockSpec((wb, dimin), lambda w: (w, 0)),
            pl.BlockSpec((1, wb), lambda w: (0, w)),
            pl.BlockSpec((1, dimin), lambda w: (0, 0)),
        ],
        out_specs=[
            pl.BlockSpec((b, wb), lambda w: (w, 0)),
            pl.BlockSpec((b, 1), lambda w: (0, 0)),
        ],
        out_shape=[
            jax.ShapeDtypeStruct((nw * b, wb), jnp.float32),
            jax.ShapeDtypeStruct((b, 1), jnp.float32),
        ],
        scratch_shapes=[pltpu.VMEM((b, wb), jnp.float32)],
        compiler_params=pltpu.CompilerParams(
            dimension_semantics=("arbitrary",),
        ),
    )(x, Ae, be, bd)


def _sc_decode(hseg, tau16, adt, bd):
    nseg_b, wb = hseg.shape
    b = tau16.shape[0]
    nseg = nseg_b // b
    width = nseg * wb
    dimin = adt.shape[1]
    rows_per = b // NWORKERS
    ncw = wb // LANES
    seg_shift = wb.bit_length() - 1
    nd = dimin // LANES
    mesh = plsc.VectorSubcoreMesh(core_axis_name="c", subcore_axis_name="s")

    @functools.partial(
        pl.kernel,
        mesh=mesh,
        out_type=jax.ShapeDtypeStruct((b * dimin,), jnp.float32),
        scratch_types=[
            pltpu.VMEM((nseg,), jnp.int32),
            pltpu.VMEM((nseg, wb), jnp.float32),
            pltpu.VMEM((width,), jnp.int32),
            pltpu.VMEM((K,), jnp.int32),
            pltpu.VMEM((K,), jnp.float32),
            pltpu.VMEM((K, dimin), jnp.float32),
            pltpu.VMEM((dimin,), jnp.float32),
            pltpu.VMEM((dimin,), jnp.float32),
            pltpu.VMEM((rows_per, LANES), jnp.float32),
            pltpu.SemaphoreType.DMA,
        ],
        compiler_params=pltpu.CompilerParams(needs_layout_passes=False),
    )
    def sc_kernel(h_hbm, tau_hbm, adt_hbm, bd_hbm, out_hbm,
                  ridx, hrow, candidx, idx64, val64, adrows, outrow, bdrow,
                  taubuf, sem):
        wid = lax.axis_index("s") * 2 + lax.axis_index("c")
        base = wid * rows_per
        pltpu.sync_copy(bd_hbm, bdrow)
        pltpu.sync_copy(tau_hbm.at[pl.ds(base, rows_per)], taubuf)

        zeros16 = jnp.zeros((LANES,), jnp.int32)
        lane_iota = lax.iota(jnp.int32, LANES)

        def do_row(i, _):
            r = base + i
            for q in range(nseg // LANES):
                ridx[pl.ds(q * LANES, LANES)] = r + b * (lane_iota + q * LANES)
            pltpu.async_copy(h_hbm.at[ridx], hrow, sem).wait()
            tau_v = taubuf[i]

            def filt_seg(sg, cnt):
                def filt(c, cnt):
                    v = hrow[sg, pl.ds(c * LANES, LANES)]
                    m = v >= tau_v
                    pref = plsc.cumsum(m.astype(jnp.int32))
                    pos = cnt + pref - 1
                    plsc.store_scatter(
                        candidx, [pos],
                        lane_iota + (sg * wb + c * LANES), mask=m)
                    return cnt + plsc.all_reduce_population_count(m)
                return lax.fori_loop(0, ncw, filt, cnt)
            cnt = lax.fori_loop(0, nseg, filt_seg, zeros16)
            c_scalar = jnp.max(cnt)
            nv = lax.shift_right_logical(c_scalar + LANES - 1, 4)
            c_splat = jnp.full((LANES,), c_scalar, jnp.int32)

            def cand_keys(j):
                ci = candidx[pl.ds(j * LANES, LANES)]
                valid = (lane_iota + j * LANES) < c_splat
                vals = plsc.load_gather(
                    hrow, [lax.shift_right_logical(ci, seg_shift),
                           lax.bitwise_and(ci, wb - 1)], mask=valid)
                vals = jnp.where(valid, vals, -jnp.inf)
                return ci, vals, valid

            def bit_step(_, carry):
                t, bit = carry
                cand_t = t | bit

                def count_step(j, acc):
                    _, vals, valid = cand_keys(j)
                    hit = valid & (_keys_of(vals) >= cand_t)
                    return acc + hit.astype(jnp.int32)
                n = jnp.sum(lax.fori_loop(0, nv, count_step, zeros16))
                t = jnp.where(n >= K, cand_t, t)
                return t, lax.shift_right_logical(bit, jnp.uint32(1))
            t_key, _ = lax.fori_loop(
                0, 32, bit_step,
                (jnp.zeros((LANES,), jnp.uint32),
                 jnp.full((LANES,), 0x80000000, jnp.uint32)))

            def min_step(j, acc):
                _, vals, valid = cand_keys(j)
                sel = valid & (_keys_of(vals) >= t_key)
                return jnp.minimum(acc, jnp.where(sel, vals, jnp.inf))
            t_f = jnp.min(lax.fori_loop(
                0, nv, min_step, jnp.full((LANES,), jnp.inf, jnp.float32)))
            t_v = jnp.full((LANES,), t_f, jnp.float32)

            def sel_pass(strict):
                def sel_step(j, cnt2):
                    ci, vals, valid = cand_keys(j)
                    if strict:
                        m = valid & (vals > t_v)
                    else:
                        m = valid & (vals == t_v)
                    pref = plsc.cumsum(m.astype(jnp.int32))
                    pos = cnt2 + pref - 1
                    mw = m & (pos < K)
                    plsc.store_scatter(idx64, [pos], ci, mask=mw)
                    plsc.store_scatter(val64, [pos], jnp.maximum(vals, 0.0),
                                       mask=mw)
                    return cnt2 + plsc.all_reduce_population_count(m)
                return sel_step
            n_gt = lax.fori_loop(0, nv, sel_pass(True), zeros16)
            lax.fori_loop(0, nv, sel_pass(False), n_gt)

            pltpu.async_copy(adt_hbm.at[idx64], adrows, sem).wait()

            for q in range(nd):
                outrow[pl.ds(q * LANES, LANES)] = bdrow[pl.ds(q * LANES,
                                                              LANES)]

            for kk in range(K // LANES):
                va = val64[pl.ds(kk * LANES, LANES)]

                def fma_step(k16, _, va=va, kk=kk):
                    vsp = _lane_splat(va, k16)
                    kidx = kk * LANES + k16
                    for q in range(nd):
                        sl = pl.ds(q * LANES, LANES)
                        outrow[sl] += vsp * adrows[kidx, sl]
                    return 0
                lax.fori_loop(0, LANES, fma_step, 0)

            pltpu.sync_copy(outrow, out_hbm.at[pl.ds(r * dimin, dimin)])
            return 0

        lax.fori_loop(0, rows_per, do_row, 0)

    return sc_kernel(hseg, tau16, adt, bd)


def kernel(x, Ae, be, Ad, bd):
    h, tau = _encode(x, Ae, be, bd)
    adt = Ad.T.reshape(Ae.shape[0], x.shape[1])
    tau16 = jnp.broadcast_to(tau, (x.shape[0], LANES))
    out = _sc_decode(h, tau16, adt, bd.reshape(-1))
    return out.reshape(x.shape[0], x.shape[1])


# speedup vs baseline: 1.2015x; 1.2015x over previous
import functools

import jax
import jax.numpy as jnp
from jax import lax
from jax.experimental import pallas as pl
from jax.experimental.pallas import tpu as pltpu
from jax.experimental.pallas import tpu_sc as plsc

K = 64
LANES = 16
NWORKERS = 32


def _lane_splat(vec, k):
    return lax.gather(
        vec, jnp.full((LANES, 1), k, jnp.int32),
        lax.GatherDimensionNumbers(offset_dims=(), collapsed_slice_dims=(0,),
                                   start_index_map=(0,)),
        (1,), mode=lax.GatherScatterMode.PROMISE_IN_BOUNDS)


def _keys_of(h):
    u = lax.bitcast_convert_type(h, jnp.uint32)
    return u ^ jnp.where(h < 0, jnp.uint32(0xFFFFFFFF), jnp.uint32(0x80000000))


def _enc_body(x_ref, ae_ref, be_ref, bd_ref, h_ref, tau_ref, gm_ref, *, nw):
    w = pl.program_id(0)
    xc = x_ref[...] - bd_ref[...]
    h = lax.dot_general(
        xc, ae_ref[...],
        dimension_numbers=(((1,), (1,)), ((), ())),
        preferred_element_type=jnp.float32,
    ) + be_ref[...]
    h_ref[...] = h

    @pl.when(w == 0)
    def _init():
        gm_ref[...] = h

    @pl.when(w > 0)
    def _acc():
        gm_ref[...] = jnp.maximum(gm_ref[...], h)

    @pl.when(w == nw - 1)
    def _threshold():
        gm = gm_ref[...]
        keys = _keys_of(gm)
        t = jnp.zeros((keys.shape[0], 1), jnp.uint32)
        for b in range(31, -1, -1):
            cand = t | jnp.uint32(1 << b)
            cnt = jnp.sum((keys >= cand).astype(jnp.int32), axis=1,
                          keepdims=True)
            t = jnp.where(cnt >= K, cand, t)
        sel = keys >= t
        tau_ref[...] = jnp.min(jnp.where(sel, gm, jnp.inf), axis=1,
                               keepdims=True)


def _encode(x, Ae, be, bd):
    b, dimin = x.shape
    width = Ae.shape[0]
    wb = 256
    nw = width // wb
    return pl.pallas_call(
        functools.partial(_enc_body, nw=nw),
        grid=(nw,),
        in_specs=[
            pl.BlockSpec((b, dimin), lambda w: (0, 0)),
            pl.BlockSpec((wb, dimin), lambda w: (w, 0)),
            pl.BlockSpec((1, wb), lambda w: (0, w)),
            pl.BlockSpec((1, dimin), lambda w: (0, 0)),
        ],
        out_specs=[
            pl.BlockSpec((b, wb), lambda w: (w, 0)),
            pl.BlockSpec((b, 1), lambda w: (0, 0)),
        ],
        out_shape=[
            jax.ShapeDtypeStruct((nw * b, wb), jnp.float32),
            jax.ShapeDtypeStruct((b, 1), jnp.float32),
        ],
        scratch_shapes=[pltpu.VMEM((b, wb), jnp.float32)],
        compiler_params=pltpu.CompilerParams(
            dimension_semantics=("arbitrary",),
        ),
    )(x, Ae, be, bd)


def _sc_decode(hseg, tau16, adt, bd):
    nseg_b, wb = hseg.shape
    b = tau16.shape[0]
    nseg = nseg_b // b
    width = nseg * wb
    dimin = adt.shape[1]
    rows_per = b // NWORKERS
    ncw = wb // LANES
    seg_shift = wb.bit_length() - 1
    nd = dimin // LANES
    mesh = plsc.VectorSubcoreMesh(core_axis_name="c", subcore_axis_name="s")

    @functools.partial(
        pl.kernel,
        mesh=mesh,
        out_type=jax.ShapeDtypeStruct((b * dimin,), jnp.float32),
        scratch_types=[
            pltpu.VMEM((nseg,), jnp.int32),
            pltpu.VMEM((nseg, wb), jnp.float32),
            pltpu.VMEM((width,), jnp.int32),
            pltpu.VMEM((K,), jnp.int32),
            pltpu.VMEM((K,), jnp.float32),
            pltpu.VMEM((K, dimin), jnp.float32),
            pltpu.VMEM((dimin,), jnp.float32),
            pltpu.VMEM((dimin,), jnp.float32),
            pltpu.VMEM((rows_per, LANES), jnp.float32),
            pltpu.SemaphoreType.DMA,
        ],
        compiler_params=pltpu.CompilerParams(needs_layout_passes=False),
    )
    def sc_kernel(h_hbm, tau_hbm, adt_hbm, bd_hbm, out_hbm,
                  ridx, hrow, candidx, idx64, val64, adrows, outrow, bdrow,
                  taubuf, sem):
        wid = lax.axis_index("s") * 2 + lax.axis_index("c")
        base = wid * rows_per
        pltpu.sync_copy(bd_hbm, bdrow)
        pltpu.sync_copy(tau_hbm.at[pl.ds(base, rows_per)], taubuf)

        zeros16 = jnp.zeros((LANES,), jnp.int32)
        lane_iota = lax.iota(jnp.int32, LANES)

        def do_row(i, _):
            r = base + i
            for q in range(nseg // LANES):
                ridx[pl.ds(q * LANES, LANES)] = r + b * (lane_iota + q * LANES)
            pltpu.async_copy(h_hbm.at[ridx], hrow, sem).wait()
            tau_v = taubuf[i]

            nstrip = (nseg * wb) // (8 * LANES)
            spseg = wb // (8 * LANES)

            def strip_loop(st, cnt):
                sg = st // spseg
                off0 = (st % spseg) * (8 * LANES)
                macc = None
                for q in range(8):
                    v = hrow[sg, pl.ds(off0 + q * LANES, LANES)]
                    m = v >= tau_v
                    macc = m if macc is None else (macc | m)

                def slow(cnt):
                    for q in range(8):
                        v = hrow[sg, pl.ds(off0 + q * LANES, LANES)]
                        m = v >= tau_v
                        pref = plsc.cumsum(m.astype(jnp.int32))
                        pos = cnt + pref - 1
                        plsc.store_scatter(
                            candidx, [pos],
                            lane_iota + (sg * wb + off0 + q * LANES), mask=m)
                        cnt = cnt + plsc.all_reduce_population_count(m)
                    return cnt
                return lax.cond(jnp.any(macc), slow, lambda c: c, cnt)
            cnt = lax.fori_loop(0, nstrip, strip_loop, zeros16)
            c_scalar = jnp.max(cnt)
            nv = lax.shift_right_logical(c_scalar + LANES - 1, 4)
            c_splat = jnp.full((LANES,), c_scalar, jnp.int32)

            def cand_keys(j):
                ci = candidx[pl.ds(j * LANES, LANES)]
                valid = (lane_iota + j * LANES) < c_splat
                vals = plsc.load_gather(
                    hrow, [lax.shift_right_logical(ci, seg_shift),
                           lax.bitwise_and(ci, wb - 1)], mask=valid)
                vals = jnp.where(valid, vals, -jnp.inf)
                return ci, vals, valid

            def bit_step(_, carry):
                t, bit = carry
                cand_t = t | bit

                def count_step(j, acc):
                    _, vals, valid = cand_keys(j)
                    hit = valid & (_keys_of(vals) >= cand_t)
                    return acc + hit.astype(jnp.int32)
                n = jnp.sum(lax.fori_loop(0, nv, count_step, zeros16))
                t = jnp.where(n >= K, cand_t, t)
                return t, lax.shift_right_logical(bit, jnp.uint32(1))
            t_key, _ = lax.fori_loop(
                0, 32, bit_step,
                (jnp.zeros((LANES,), jnp.uint32),
                 jnp.full((LANES,), 0x80000000, jnp.uint32)))

            def min_step(j, acc):
                _, vals, valid = cand_keys(j)
                sel = valid & (_keys_of(vals) >= t_key)
                return jnp.minimum(acc, jnp.where(sel, vals, jnp.inf))
            t_f = jnp.min(lax.fori_loop(
                0, nv, min_step, jnp.full((LANES,), jnp.inf, jnp.float32)))
            t_v = jnp.full((LANES,), t_f, jnp.float32)

            def sel_pass(strict):
                def sel_step(j, cnt2):
                    ci, vals, valid = cand_keys(j)
                    if strict:
                        m = valid & (vals > t_v)
                    else:
                        m = valid & (vals == t_v)
                    pref = plsc.cumsum(m.astype(jnp.int32))
                    pos = cnt2 + pref - 1
                    mw = m & (pos < K)
                    plsc.store_scatter(idx64, [pos], ci, mask=mw)
                    plsc.store_scatter(val64, [pos], jnp.maximum(vals, 0.0),
                                       mask=mw)
                    return cnt2 + plsc.all_reduce_population_count(m)
                return sel_step
            n_gt = lax.fori_loop(0, nv, sel_pass(True), zeros16)
            lax.fori_loop(0, nv, sel_pass(False), n_gt)

            pltpu.async_copy(adt_hbm.at[idx64], adrows, sem).wait()

            for q in range(nd):
                outrow[pl.ds(q * LANES, LANES)] = bdrow[pl.ds(q * LANES,
                                                              LANES)]

            for kk in range(K // LANES):
                va = val64[pl.ds(kk * LANES, LANES)]

                def fma_step(k16, _, va=va, kk=kk):
                    vsp = _lane_splat(va, k16)
                    kidx = kk * LANES + k16
                    for q in range(nd):
                        sl = pl.ds(q * LANES, LANES)
                        outrow[sl] += vsp * adrows[kidx, sl]
                    return 0
                lax.fori_loop(0, LANES, fma_step, 0)

            pltpu.sync_copy(outrow, out_hbm.at[pl.ds(r * dimin, dimin)])
            return 0

        lax.fori_loop(0, rows_per, do_row, 0)

    return sc_kernel(hseg, tau16, adt, bd)


def kernel(x, Ae, be, Ad, bd):
    h, tau = _encode(x, Ae, be, bd)
    adt = Ad.T.reshape(Ae.shape[0], x.shape[1])
    tau16 = jnp.broadcast_to(tau, (x.shape[0], LANES))
    out = _sc_decode(h, tau16, adt, bd.reshape(-1))
    return out.reshape(x.shape[0], x.shape[1])


# speedup vs baseline: 2.0415x; 1.6991x over previous
import functools

import jax
import jax.numpy as jnp
from jax import lax
from jax.experimental import pallas as pl
from jax.experimental.pallas import tpu as pltpu
from jax.experimental.pallas import tpu_sc as plsc

K = 64
LANES = 16
NWORKERS = 32


def _lane_splat(vec, k):
    return lax.gather(
        vec, jnp.full((LANES, 1), k, jnp.int32),
        lax.GatherDimensionNumbers(offset_dims=(), collapsed_slice_dims=(0,),
                                   start_index_map=(0,)),
        (1,), mode=lax.GatherScatterMode.PROMISE_IN_BOUNDS)


def _keys_of(h):
    u = lax.bitcast_convert_type(h, jnp.uint32)
    return u ^ jnp.where(h < 0, jnp.uint32(0xFFFFFFFF), jnp.uint32(0x80000000))


def _enc_body(x_ref, ae_ref, be_ref, bd_ref, h_ref, tau_ref, gm_ref, *, nw):
    w = pl.program_id(0)
    xc = x_ref[...] - bd_ref[...]
    h = lax.dot_general(
        xc, ae_ref[...],
        dimension_numbers=(((1,), (1,)), ((), ())),
        preferred_element_type=jnp.float32,
    ) + be_ref[...]
    h_ref[...] = h

    @pl.when(w == 0)
    def _init():
        gm_ref[...] = h

    @pl.when(w > 0)
    def _acc():
        gm_ref[...] = jnp.maximum(gm_ref[...], h)

    @pl.when(w == nw - 1)
    def _threshold():
        gm = gm_ref[...]
        keys = _keys_of(gm)
        t = jnp.zeros((keys.shape[0], 1), jnp.uint32)
        for b in range(31, -1, -1):
            cand = t | jnp.uint32(1 << b)
            cnt = jnp.sum((keys >= cand).astype(jnp.int32), axis=1,
                          keepdims=True)
            t = jnp.where(cnt >= K, cand, t)
        sel = keys >= t
        tau_ref[...] = jnp.min(jnp.where(sel, gm, jnp.inf), axis=1,
                               keepdims=True)


def _encode(x, Ae, be, bd):
    b, dimin = x.shape
    width = Ae.shape[0]
    wb = 256
    nw = width // wb
    return pl.pallas_call(
        functools.partial(_enc_body, nw=nw),
        grid=(nw,),
        in_specs=[
            pl.BlockSpec((b, dimin), lambda w: (0, 0)),
            pl.BlockSpec((wb, dimin), lambda w: (w, 0)),
            pl.BlockSpec((1, wb), lambda w: (0, w)),
            pl.BlockSpec((1, dimin), lambda w: (0, 0)),
        ],
        out_specs=[
            pl.BlockSpec((b, wb), lambda w: (w, 0)),
            pl.BlockSpec((b, 1), lambda w: (0, 0)),
        ],
        out_shape=[
            jax.ShapeDtypeStruct((nw * b, wb), jnp.float32),
            jax.ShapeDtypeStruct((b, 1), jnp.float32),
        ],
        scratch_shapes=[pltpu.VMEM((b, wb), jnp.float32)],
        compiler_params=pltpu.CompilerParams(
            dimension_semantics=("arbitrary",),
        ),
    )(x, Ae, be, bd)


def _sc_decode(hseg, tau16, adt, bd):
    nseg_b, wb = hseg.shape
    b = tau16.shape[0]
    nseg = nseg_b // b
    width = nseg * wb
    dimin = adt.shape[1]
    rows_per = b // NWORKERS
    ncw = wb // LANES
    seg_shift = wb.bit_length() - 1
    nd = dimin // LANES
    mesh = plsc.VectorSubcoreMesh(core_axis_name="c", subcore_axis_name="s")

    @functools.partial(
        pl.kernel,
        mesh=mesh,
        out_type=jax.ShapeDtypeStruct((b * dimin,), jnp.float32),
        scratch_types=[
            pltpu.VMEM((nseg,), jnp.int32),
            pltpu.VMEM((nseg, wb), jnp.float32),
            pltpu.VMEM((width,), jnp.int32),
            pltpu.VMEM((K,), jnp.int32),
            pltpu.VMEM((K,), jnp.float32),
            pltpu.VMEM((K, dimin), jnp.float32),
            pltpu.VMEM((dimin,), jnp.float32),
            pltpu.VMEM((dimin,), jnp.float32),
            pltpu.VMEM((rows_per, LANES), jnp.float32),
            pltpu.SemaphoreType.DMA,
        ],
        compiler_params=pltpu.CompilerParams(needs_layout_passes=False),
    )
    def sc_kernel(h_hbm, tau_hbm, adt_hbm, bd_hbm, out_hbm,
                  ridx, hrow, candidx, idx64, val64, adrows, outrow, bdrow,
                  taubuf, sem):
        wid = lax.axis_index("s") * 2 + lax.axis_index("c")
        base = wid * rows_per
        pltpu.sync_copy(bd_hbm, bdrow)
        pltpu.sync_copy(tau_hbm.at[pl.ds(base, rows_per)], taubuf)

        zeros16 = jnp.zeros((LANES,), jnp.int32)
        lane_iota = lax.iota(jnp.int32, LANES)

        def do_row(i, _):
            r = base + i
            for q in range(nseg // LANES):
                ridx[pl.ds(q * LANES, LANES)] = r + b * (lane_iota + q * LANES)
            pltpu.async_copy(h_hbm.at[ridx], hrow, sem).wait()
            tau_v = taubuf[i]

            nstrip = (nseg * wb) // (8 * LANES)
            spseg = wb // (8 * LANES)

            def strip_loop(st, cnt):
                sg = st // spseg
                off0 = (st % spseg) * (8 * LANES)
                macc = None
                for q in range(8):
                    v = hrow[sg, pl.ds(off0 + q * LANES, LANES)]
                    m = v >= tau_v
                    macc = m if macc is None else (macc | m)

                def slow(cnt):
                    for q in range(8):
                        v = hrow[sg, pl.ds(off0 + q * LANES, LANES)]
                        m = v >= tau_v
                        pref = plsc.cumsum(m.astype(jnp.int32))
                        pos = cnt + pref - 1
                        plsc.store_scatter(
                            candidx, [pos],
                            lane_iota + (sg * wb + off0 + q * LANES), mask=m)
                        cnt = cnt + plsc.all_reduce_population_count(m)
                    return cnt
                return lax.cond(jnp.any(macc), slow, lambda c: c, cnt)
            cnt = lax.fori_loop(0, nstrip, strip_loop, zeros16)
            c_scalar = jnp.max(cnt)
            nv = lax.shift_right_logical(c_scalar + LANES - 1, 4)
            c_splat = jnp.full((LANES,), c_scalar, jnp.int32)

            def cand_keys(j):
                ci = candidx[pl.ds(j * LANES, LANES)]
                valid = (lane_iota + j * LANES) < c_splat
                vals = plsc.load_gather(
                    hrow, [lax.shift_right_logical(ci, seg_shift),
                           lax.bitwise_and(ci, wb - 1)], mask=valid)
                vals = jnp.where(valid, vals, -jnp.inf)
                return ci, vals, valid

            def bit_step(_, carry):
                t, bit = carry
                cand_t = t | bit

                def count_step(j, acc):
                    _, vals, valid = cand_keys(j)
                    hit = valid & (_keys_of(vals) >= cand_t)
                    return acc + hit.astype(jnp.int32)
                n = jnp.sum(lax.fori_loop(0, nv, count_step, zeros16))
                t = jnp.where(n >= K, cand_t, t)
                return t, lax.shift_right_logical(bit, jnp.uint32(1))
            t_key, _ = lax.fori_loop(
                0, 32, bit_step,
                (jnp.zeros((LANES,), jnp.uint32),
                 jnp.full((LANES,), 0x80000000, jnp.uint32)))

            def min_step(j, acc):
                _, vals, valid = cand_keys(j)
                sel = valid & (_keys_of(vals) >= t_key)
                return jnp.minimum(acc, jnp.where(sel, vals, jnp.inf))
            t_f = jnp.min(lax.fori_loop(
                0, nv, min_step, jnp.full((LANES,), jnp.inf, jnp.float32)))
            t_v = jnp.full((LANES,), t_f, jnp.float32)

            def sel_pass(strict):
                def sel_step(j, cnt2):
                    ci, vals, valid = cand_keys(j)
                    if strict:
                        m = valid & (vals > t_v)
                    else:
                        m = valid & (vals == t_v)
                    pref = plsc.cumsum(m.astype(jnp.int32))
                    pos = cnt2 + pref - 1
                    mw = m & (pos < K)
                    plsc.store_scatter(idx64, [pos], ci, mask=mw)
                    plsc.store_scatter(val64, [pos], jnp.maximum(vals, 0.0),
                                       mask=mw)
                    return cnt2 + plsc.all_reduce_population_count(m)
                return sel_step
            n_gt = lax.fori_loop(0, nv, sel_pass(True), zeros16)
            lax.fori_loop(0, nv, sel_pass(False), n_gt)

            pltpu.async_copy(adt_hbm.at[idx64], adrows, sem).wait()

            ngrp = 4
            gsz = nd // ngrp
            for g in range(ngrp):
                accs = tuple(
                    bdrow[pl.ds((g * gsz + q) * LANES, LANES)]
                    for q in range(gsz))
                for kk in range(K // LANES):
                    va = val64[pl.ds(kk * LANES, LANES)]

                    def fma_step(k16, accs, va=va, kk=kk, g=g):
                        vsp = _lane_splat(va, k16)
                        kidx = kk * LANES + k16
                        return tuple(
                            accs[q] + vsp * adrows[
                                kidx, pl.ds((g * gsz + q) * LANES, LANES)]
                            for q in range(gsz))
                    accs = lax.fori_loop(0, LANES, fma_step, accs)
                for q in range(gsz):
                    outrow[pl.ds((g * gsz + q) * LANES, LANES)] = accs[q]

            pltpu.sync_copy(outrow, out_hbm.at[pl.ds(r * dimin, dimin)])
            return 0

        lax.fori_loop(0, rows_per, do_row, 0)

    return sc_kernel(hseg, tau16, adt, bd)


def kernel(x, Ae, be, Ad, bd):
    h, tau = _encode(x, Ae, be, bd)
    adt = Ad.T.reshape(Ae.shape[0], x.shape[1])
    tau16 = jnp.broadcast_to(tau, (x.shape[0], LANES))
    out = _sc_decode(h, tau16, adt, bd.reshape(-1))
    return out.reshape(x.shape[0], x.shape[1])


# speedup vs baseline: 2.0416x; 1.0000x over previous
import functools

import jax
import jax.numpy as jnp
from jax import lax
from jax.experimental import pallas as pl
from jax.experimental.pallas import tpu as pltpu
from jax.experimental.pallas import tpu_sc as plsc

K = 64
LANES = 16
NWORKERS = 32


def _lane_splat(vec, k):
    return lax.gather(
        vec, jnp.full((LANES, 1), k, jnp.int32),
        lax.GatherDimensionNumbers(offset_dims=(), collapsed_slice_dims=(0,),
                                   start_index_map=(0,)),
        (1,), mode=lax.GatherScatterMode.PROMISE_IN_BOUNDS)


def _keys_of(h):
    u = lax.bitcast_convert_type(h, jnp.uint32)
    return u ^ jnp.where(h < 0, jnp.uint32(0xFFFFFFFF), jnp.uint32(0x80000000))


def _enc_body(x_ref, ae_ref, be_ref, bd_ref, h_ref, tau_ref, gm_ref, *, nw):
    w = pl.program_id(0)
    xc = x_ref[...] - bd_ref[...]
    h = lax.dot_general(
        xc, ae_ref[...],
        dimension_numbers=(((1,), (1,)), ((), ())),
        preferred_element_type=jnp.float32,
    ) + be_ref[...]
    h_ref[...] = h

    @pl.when(w == 0)
    def _init():
        gm_ref[...] = h

    @pl.when(w > 0)
    def _acc():
        gm_ref[...] = jnp.maximum(gm_ref[...], h)

    @pl.when(w == nw - 1)
    def _threshold():
        gm = gm_ref[...]
        keys = _keys_of(gm)
        t = jnp.zeros((keys.shape[0], 1), jnp.uint32)
        for b in range(31, -1, -1):
            cand = t | jnp.uint32(1 << b)
            cnt = jnp.sum((keys >= cand).astype(jnp.int32), axis=1,
                          keepdims=True)
            t = jnp.where(cnt >= K, cand, t)
        sel = keys >= t
        tau_ref[...] = jnp.min(jnp.where(sel, gm, jnp.inf), axis=1,
                               keepdims=True)


def _encode(x, Ae, be, bd):
    b, dimin = x.shape
    width = Ae.shape[0]
    wb = 256
    nw = width // wb
    return pl.pallas_call(
        functools.partial(_enc_body, nw=nw),
        grid=(nw,),
        in_specs=[
            pl.BlockSpec((b, dimin), lambda w: (0, 0)),
            pl.BlockSpec((wb, dimin), lambda w: (w, 0)),
            pl.BlockSpec((1, wb), lambda w: (0, w)),
            pl.BlockSpec((1, dimin), lambda w: (0, 0)),
        ],
        out_specs=[
            pl.BlockSpec((b, wb), lambda w: (w, 0)),
            pl.BlockSpec((b, 1), lambda w: (0, 0)),
        ],
        out_shape=[
            jax.ShapeDtypeStruct((nw * b, wb), jnp.float32),
            jax.ShapeDtypeStruct((b, 1), jnp.float32),
        ],
        scratch_shapes=[pltpu.VMEM((b, wb), jnp.float32)],
        compiler_params=pltpu.CompilerParams(
            dimension_semantics=("arbitrary",),
        ),
    )(x, Ae, be, bd)


def _sc_decode(hseg, tau16, adt, bd):
    nseg_b, wb = hseg.shape
    b = tau16.shape[0]
    nseg = nseg_b // b
    width = nseg * wb
    dimin = adt.shape[1]
    rows_per = b // NWORKERS
    ncw = wb // LANES
    seg_shift = wb.bit_length() - 1
    nd = dimin // LANES
    mesh = plsc.VectorSubcoreMesh(core_axis_name="c", subcore_axis_name="s")

    @functools.partial(
        pl.kernel,
        mesh=mesh,
        out_type=jax.ShapeDtypeStruct((b * dimin,), jnp.float32),
        scratch_types=[
            pltpu.VMEM((nseg,), jnp.int32),
            pltpu.VMEM((nseg, wb), jnp.float32),
            pltpu.VMEM((width,), jnp.int32),
            pltpu.VMEM((K,), jnp.int32),
            pltpu.VMEM((K,), jnp.float32),
            pltpu.VMEM((K // 2, dimin), jnp.float32),
            pltpu.VMEM((K // 2, dimin), jnp.float32),
            pltpu.VMEM((dimin,), jnp.float32),
            pltpu.VMEM((dimin,), jnp.float32),
            pltpu.VMEM((rows_per, LANES), jnp.float32),
            pltpu.SemaphoreType.DMA,
            pltpu.SemaphoreType.DMA,
        ],
        compiler_params=pltpu.CompilerParams(needs_layout_passes=False),
    )
    def sc_kernel(h_hbm, tau_hbm, adt_hbm, bd_hbm, out_hbm,
                  ridx, hrow, candidx, idx64, val64, adrows_a, adrows_b,
                  outrow, bdrow, taubuf, sem, sem_b):
        wid = lax.axis_index("s") * 2 + lax.axis_index("c")
        base = wid * rows_per
        pltpu.sync_copy(bd_hbm, bdrow)
        pltpu.sync_copy(tau_hbm.at[pl.ds(base, rows_per)], taubuf)

        zeros16 = jnp.zeros((LANES,), jnp.int32)
        lane_iota = lax.iota(jnp.int32, LANES)

        def do_row(i, _):
            r = base + i
            for q in range(nseg // LANES):
                ridx[pl.ds(q * LANES, LANES)] = r + b * (lane_iota + q * LANES)
            pltpu.async_copy(h_hbm.at[ridx], hrow, sem).wait()
            tau_v = taubuf[i]

            nstrip = (nseg * wb) // (8 * LANES)
            spseg = wb // (8 * LANES)

            def strip_loop(st, cnt):
                sg = st // spseg
                off0 = (st % spseg) * (8 * LANES)
                macc = None
                for q in range(8):
                    v = hrow[sg, pl.ds(off0 + q * LANES, LANES)]
                    m = v >= tau_v
                    macc = m if macc is None else (macc | m)

                def slow(cnt):
                    for q in range(8):
                        v = hrow[sg, pl.ds(off0 + q * LANES, LANES)]
                        m = v >= tau_v
                        pref = plsc.cumsum(m.astype(jnp.int32))
                        pos = cnt + pref - 1
                        plsc.store_scatter(
                            candidx, [pos],
                            lane_iota + (sg * wb + off0 + q * LANES), mask=m)
                        cnt = cnt + plsc.all_reduce_population_count(m)
                    return cnt
                return lax.cond(jnp.any(macc), slow, lambda c: c, cnt)
            cnt = lax.fori_loop(0, nstrip, strip_loop, zeros16)
            c_scalar = jnp.max(cnt)
            nv = lax.shift_right_logical(c_scalar + LANES - 1, 4)
            c_splat = jnp.full((LANES,), c_scalar, jnp.int32)

            def cand_keys(j):
                ci = candidx[pl.ds(j * LANES, LANES)]
                valid = (lane_iota + j * LANES) < c_splat
                vals = plsc.load_gather(
                    hrow, [lax.shift_right_logical(ci, seg_shift),
                           lax.bitwise_and(ci, wb - 1)], mask=valid)
                vals = jnp.where(valid, vals, -jnp.inf)
                return ci, vals, valid

            def bit_step(_, carry):
                t, bit = carry
                cand_t = t | bit

                def count_step(j, acc):
                    _, vals, valid = cand_keys(j)
                    hit = valid & (_keys_of(vals) >= cand_t)
                    return acc + hit.astype(jnp.int32)
                n = jnp.sum(lax.fori_loop(0, nv, count_step, zeros16))
                t = jnp.where(n >= K, cand_t, t)
                return t, lax.shift_right_logical(bit, jnp.uint32(1))
            t_key, _ = lax.fori_loop(
                0, 32, bit_step,
                (jnp.zeros((LANES,), jnp.uint32),
                 jnp.full((LANES,), 0x80000000, jnp.uint32)))

            def min_step(j, acc):
                _, vals, valid = cand_keys(j)
                sel = valid & (_keys_of(vals) >= t_key)
                return jnp.minimum(acc, jnp.where(sel, vals, jnp.inf))
            t_f = jnp.min(lax.fori_loop(
                0, nv, min_step, jnp.full((LANES,), jnp.inf, jnp.float32)))
            t_v = jnp.full((LANES,), t_f, jnp.float32)

            def sel_pass(strict):
                def sel_step(j, cnt2):
                    ci, vals, valid = cand_keys(j)
                    if strict:
                        m = valid & (vals > t_v)
                    else:
                        m = valid & (vals == t_v)
                    pref = plsc.cumsum(m.astype(jnp.int32))
                    pos = cnt2 + pref - 1
                    mw = m & (pos < K)
                    plsc.store_scatter(idx64, [pos], ci, mask=mw)
                    plsc.store_scatter(val64, [pos], jnp.maximum(vals, 0.0),
                                       mask=mw)
                    return cnt2 + plsc.all_reduce_population_count(m)
                return sel_step
            n_gt = lax.fori_loop(0, nv, sel_pass(True), zeros16)
            lax.fori_loop(0, nv, sel_pass(False), n_gt)

            cp_a = pltpu.async_copy(
                adt_hbm.at[idx64.at[pl.ds(0, K // 2)]], adrows_a, sem)
            cp_b = pltpu.async_copy(
                adt_hbm.at[idx64.at[pl.ds(K // 2, K // 2)]], adrows_b, sem_b)
            cp_a.wait()

            ngrp = 4
            gsz = nd // ngrp
            for g in range(ngrp):
                accs = tuple(
                    bdrow[pl.ds((g * gsz + q) * LANES, LANES)]
                    for q in range(gsz))
                for kk in range(K // LANES):
                    if g == 0 and kk == (K // LANES) // 2:
                        cp_b.wait()
                    va = val64[pl.ds(kk * LANES, LANES)]
                    half = kk < (K // LANES) // 2
                    adrows = adrows_a if half else adrows_b

                    def fma_step(k16, accs, va=va, kk=kk, g=g, adrows=adrows):
                        vsp = _lane_splat(va, k16)
                        kidx = (kk % ((K // LANES) // 2)) * LANES + k16
                        return tuple(
                            accs[q] + vsp * adrows[
                                kidx, pl.ds((g * gsz + q) * LANES, LANES)]
                            for q in range(gsz))
                    accs = lax.fori_loop(0, LANES, fma_step, accs)
                for q in range(gsz):
                    outrow[pl.ds((g * gsz + q) * LANES, LANES)] = accs[q]

            pltpu.sync_copy(outrow, out_hbm.at[pl.ds(r * dimin, dimin)])
            return 0

        lax.fori_loop(0, rows_per, do_row, 0)

    return sc_kernel(hseg, tau16, adt, bd)


def kernel(x, Ae, be, Ad, bd):
    h, tau = _encode(x, Ae, be, bd)
    adt = Ad.T.reshape(Ae.shape[0], x.shape[1])
    tau16 = jnp.broadcast_to(tau, (x.shape[0], LANES))
    out = _sc_decode(h, tau16, adt, bd.reshape(-1))
    return out.reshape(x.shape[0], x.shape[1])
